# baseline (speedup 1.0000x reference)
import functools

import jax
import jax.numpy as jnp
from jax import lax
from jax.experimental import pallas as pl
from jax.experimental.pallas import tpu as pltpu
from jax.experimental.pallas import tpu_sc as plsc

N = 100000
E = 3200000
DIN = 128
DOUT = 16
DH = DOUT // 2

NC = 2
NS = 16

ROW_BLK = 2048
N_PAD = 100352
TC_GRID = N_PAD // ROW_BLK
SEG = N_PAD // NS
ZCH = SEG // 8

E_PAD = 2048 * 32 * 50
IDXN = 2048
EPT = E_PAD // NS
NCHUNK = EPT // IDXN
DEPT = E_PAD // (NC * NS)
DNCHUNK = DEPT // IDXN

_mesh = plsc.VectorSubcoreMesh(
    core_axis_name="c", subcore_axis_name="s", num_cores=NC, num_subcores=NS
)
_sc_params = pltpu.CompilerParams(use_tc_tiling_on_sc=False)


def _fill1d(ref, nvec, value):
  v = jnp.full((16,), value, jnp.float32)

  def body(i, _):
    ref[pl.ds(i * 16, 16)] = v
    return 0

  lax.fori_loop(0, nvec, body, 0)


def _fill2d(ref, nvec, value):
  v = jnp.full((2, 8), value, jnp.float32)

  def body(i, _):
    ref[pl.ds(i * 2, 2), :] = v
    return 0

  lax.fori_loop(0, nvec, body, 0)


@functools.partial(
    pl.kernel,
    out_type=jax.ShapeDtypeStruct((NC, N_PAD), jnp.float32),
    mesh=_mesh,
    scratch_types=[
        pltpu.VMEM((IDXN,), jnp.int32),
        pltpu.VMEM((IDXN,), jnp.float32),
        pltpu.VMEM((SEG,), jnp.float32),
        pltpu.VMEM_SHARED((N_PAD,), jnp.float32),
    ],
    compiler_params=_sc_params,
)
def _sc_degree(dst_hbm, deg_hbm, idx_v, ones_v, stage_v, deg_sp):
  cid = lax.axis_index("c")
  sid = lax.axis_index("s")
  wid = cid * NS + sid

  _fill1d(ones_v, IDXN // 16, 1.0)
  _fill1d(stage_v, SEG // 16, 0.0)
  pltpu.sync_copy(stage_v, deg_sp.at[pl.ds(sid * SEG, SEG)])
  plsc.subcore_barrier()

  tbase = wid * DEPT

  def chunk_body(ci, _):
    pltpu.sync_copy(dst_hbm.at[pl.ds(tbase + ci * IDXN, IDXN)], idx_v)
    pltpu.sync_copy(ones_v, deg_sp.at[idx_v], add=True)
    return 0

  lax.fori_loop(0, DNCHUNK, chunk_body, 0)
  plsc.subcore_barrier()

  pltpu.sync_copy(deg_sp.at[pl.ds(sid * SEG, SEG)],
                  deg_hbm.at[cid, pl.ds(sid * SEG, SEG)])


@functools.partial(
    pl.kernel,
    out_type=jax.ShapeDtypeStruct((NC, N_PAD, DH), jnp.float32),
    mesh=_mesh,
    scratch_types=[
        pltpu.VMEM((IDXN,), jnp.int32),
        pltpu.VMEM((IDXN,), jnp.int32),
        pltpu.VMEM((IDXN, DH), jnp.float32),
        pltpu.VMEM((ZCH, DH), jnp.float32),
        pltpu.VMEM_SHARED((N_PAD, DH), jnp.float32),
        pltpu.SemaphoreType.DMA,
    ],
    compiler_params=_sc_params,
)
def _sc_agg(hs0_hbm, hs1_hbm, src_hbm, dst_hbm, acc_hbm,
            src_v, dst_v, rows_v, stage_v, acc_sp, sem):
  cid = lax.axis_index("c")
  sid = lax.axis_index("s")

  _fill2d(stage_v, ZCH // 2, 0.0)
  for k in range(8):
    pltpu.sync_copy(stage_v, acc_sp.at[pl.ds(sid * SEG + k * ZCH, ZCH)])
  plsc.subcore_barrier()

  tbase = sid * EPT

  def chunk_core(ci, hs_hbm):
    r0 = tbase + ci * IDXN
    pltpu.sync_copy(src_hbm.at[pl.ds(r0, IDXN)], src_v)
    pltpu.sync_copy(dst_hbm.at[pl.ds(r0, IDXN)], dst_v)
    pltpu.async_copy(hs_hbm.at[src_v], rows_v, sem).wait()
    pltpu.sync_copy(rows_v, acc_sp.at[dst_v], add=True)

  @pl.when(cid == 0)
  def _():
    def body(ci, _):
      chunk_core(ci, hs0_hbm)
      return 0
    lax.fori_loop(0, NCHUNK, body, 0)

  @pl.when(cid == 1)
  def _():
    def body(ci, _):
      chunk_core(ci, hs1_hbm)
      return 0
    lax.fori_loop(0, NCHUNK, body, 0)

  plsc.subcore_barrier()

  pltpu.sync_copy(acc_sp.at[pl.ds(sid * SEG, SEG)],
                  acc_hbm.at[cid, pl.ds(sid * SEG, SEG)])


def _tc_z_body(x_ref, w_ref, z_ref):
  z_ref[...] = jnp.dot(x_ref[...], w_ref[...],
                       preferred_element_type=jnp.float32)


def _tc_scale_body(z_ref, d0_ref, d1_ref, hs0_ref, hs1_ref, dis_ref):
  d = d0_ref[...] + d1_ref[...] + 1.0
  dis = jnp.where(d > 0, lax.rsqrt(jnp.maximum(d, 1e-12)), 0.0)
  hs = z_ref[...] * dis
  hs0_ref[...] = hs[:, :DH]
  hs1_ref[...] = hs[:, DH:]
  dis_ref[...] = dis


def _tc_mid_body(a0_ref, a1_ref, hs0_ref, hs1_ref, dis_ref, w_ref, b_ref,
                 o0_ref, o1_ref):
  dis = dis_ref[...]
  a = jnp.concatenate([a0_ref[...], a1_ref[...]], axis=1)
  hs = jnp.concatenate([hs0_ref[...], hs1_ref[...]], axis=1)
  h = dis * (a + hs) + b_ref[...]
  h = jnp.maximum(h, 0.0)
  z = jnp.dot(h, w_ref[...], preferred_element_type=jnp.float32) * dis
  o0_ref[...] = z[:, :DH]
  o1_ref[...] = z[:, DH:]


def _tc_last_body(a0_ref, a1_ref, hs0_ref, hs1_ref, dis_ref, b_ref, out_ref):
  a = jnp.concatenate([a0_ref[...], a1_ref[...]], axis=1)
  hs = jnp.concatenate([hs0_ref[...], hs1_ref[...]], axis=1)
  out_ref[...] = dis_ref[...] * (a + hs) + b_ref[...]


def _rows_spec(width):
  return pl.BlockSpec((ROW_BLK, width), lambda i: (i, 0))


def _full_spec(shape):
  return pl.BlockSpec(shape, lambda i: tuple(0 for _ in shape))


_tc_params = pltpu.CompilerParams(dimension_semantics=("arbitrary",))

_tc_z = pl.pallas_call(
    _tc_z_body,
    grid=(TC_GRID,),
    in_specs=[_rows_spec(DIN), _full_spec((DIN, DOUT))],
    out_specs=_rows_spec(DOUT),
    out_shape=jax.ShapeDtypeStruct((N_PAD, DOUT), jnp.float32),
    compiler_params=_tc_params,
)

_tc_scale = pl.pallas_call(
    _tc_scale_body,
    grid=(TC_GRID,),
    in_specs=[_rows_spec(DOUT), _rows_spec(1), _rows_spec(1)],
    out_specs=[_rows_spec(DH), _rows_spec(DH), _rows_spec(1)],
    out_shape=[
        jax.ShapeDtypeStruct((N_PAD, DH), jnp.float32),
        jax.ShapeDtypeStruct((N_PAD, DH), jnp.float32),
        jax.ShapeDtypeStruct((N_PAD, 1), jnp.float32),
    ],
    compiler_params=_tc_params,
)

_tc_mid = pl.pallas_call(
    _tc_mid_body,
    grid=(TC_GRID,),
    in_specs=[
        _rows_spec(DH),
        _rows_spec(DH),
        _rows_spec(DH),
        _rows_spec(DH),
        _rows_spec(1),
        _full_spec((DOUT, DOUT)),
        _full_spec((1, DOUT)),
    ],
    out_specs=[_rows_spec(DH), _rows_spec(DH)],
    out_shape=[
        jax.ShapeDtypeStruct((N_PAD, DH), jnp.float32),
        jax.ShapeDtypeStruct((N_PAD, DH), jnp.float32),
    ],
    compiler_params=_tc_params,
)

_tc_last = pl.pallas_call(
    _tc_last_body,
    grid=(TC_GRID,),
    in_specs=[
        _rows_spec(DH),
        _rows_spec(DH),
        _rows_spec(DH),
        _rows_spec(DH),
        _rows_spec(1),
        _full_spec((1, DOUT)),
    ],
    out_specs=_rows_spec(DOUT),
    out_shape=jax.ShapeDtypeStruct((N_PAD, DOUT), jnp.float32),
    compiler_params=_tc_params,
)


@jax.jit
def kernel(x, edge_index, W1, b1, W2, b2):
  pad_e = E_PAD - E
  src = jnp.concatenate([edge_index[0], jnp.full((pad_e,), N, jnp.int32)])
  dst = jnp.concatenate([edge_index[1], jnp.full((pad_e,), N, jnp.int32)])
  x_pad = jnp.pad(x, ((0, N_PAD - N), (0, 0)))

  deg = _sc_degree(dst)
  z1 = _tc_z(x_pad, W1)
  d0 = deg[0].reshape(N_PAD, 1)
  d1 = deg[1].reshape(N_PAD, 1)

  hs1a, hs1b, dis = _tc_scale(z1, d0, d1)

  acc1 = _sc_agg(hs1a, hs1b, src, dst)
  hs2a, hs2b = _tc_mid(acc1[0], acc1[1], hs1a, hs1b, dis, W2,
                       b1.reshape(1, DOUT))

  acc2 = _sc_agg(hs2a, hs2b, src, dst)
  out = _tc_last(acc2[0], acc2[1], hs2a, hs2b, dis, b2.reshape(1, DOUT))
  return out[:N]


# speedup vs baseline: 1.0542x; 1.0542x over previous
import functools

import jax
import jax.numpy as jnp
from jax import lax
from jax.experimental import pallas as pl
from jax.experimental.pallas import tpu as pltpu
from jax.experimental.pallas import tpu_sc as plsc

N = 100000
E = 3200000
DIN = 128
DOUT = 16
DH = DOUT // 2

NC = 2
NS = 16

ROW_BLK = 2048
N_PAD = 100352
TC_GRID = N_PAD // ROW_BLK
SEG = N_PAD // NS
ZCH = SEG // 8

E_PAD = 2048 * 32 * 50
IDXN = 4096
EPT = E_PAD // NS
NCHUNK = EPT // IDXN
DEPT = E_PAD // (NC * NS)
DNCHUNK = DEPT // IDXN

_mesh = plsc.VectorSubcoreMesh(
    core_axis_name="c", subcore_axis_name="s", num_cores=NC, num_subcores=NS
)
_sc_params = pltpu.CompilerParams(use_tc_tiling_on_sc=False)


def _fill1d(ref, nvec, value):
  v = jnp.full((16,), value, jnp.float32)

  def body(i, _):
    ref[pl.ds(i * 16, 16)] = v
    return 0

  lax.fori_loop(0, nvec, body, 0)


def _fill2d(ref, nvec, value):
  v = jnp.full((2, 8), value, jnp.float32)

  def body(i, _):
    ref[pl.ds(i * 2, 2), :] = v
    return 0

  lax.fori_loop(0, nvec, body, 0)


@functools.partial(
    pl.kernel,
    out_type=jax.ShapeDtypeStruct((NC, N_PAD), jnp.float32),
    mesh=_mesh,
    scratch_types=[
        pltpu.VMEM((IDXN,), jnp.int32),
        pltpu.VMEM((IDXN,), jnp.float32),
        pltpu.VMEM((SEG,), jnp.float32),
        pltpu.VMEM_SHARED((N_PAD,), jnp.float32),
    ],
    compiler_params=_sc_params,
)
def _sc_degree(dst_hbm, deg_hbm, idx_v, ones_v, stage_v, deg_sp):
  cid = lax.axis_index("c")
  sid = lax.axis_index("s")
  wid = cid * NS + sid

  _fill1d(ones_v, IDXN // 16, 1.0)
  _fill1d(stage_v, SEG // 16, 0.0)
  pltpu.sync_copy(stage_v, deg_sp.at[pl.ds(sid * SEG, SEG)])
  plsc.subcore_barrier()

  tbase = wid * DEPT

  def chunk_body(ci, _):
    pltpu.sync_copy(dst_hbm.at[pl.ds(tbase + ci * IDXN, IDXN)], idx_v)
    pltpu.sync_copy(ones_v, deg_sp.at[idx_v], add=True)
    return 0

  lax.fori_loop(0, DNCHUNK, chunk_body, 0)
  plsc.subcore_barrier()

  pltpu.sync_copy(deg_sp.at[pl.ds(sid * SEG, SEG)],
                  deg_hbm.at[cid, pl.ds(sid * SEG, SEG)])


@functools.partial(
    pl.kernel,
    out_type=jax.ShapeDtypeStruct((NC, N_PAD, DH), jnp.float32),
    mesh=_mesh,
    scratch_types=[
        pltpu.VMEM((IDXN,), jnp.int32),
        pltpu.VMEM((IDXN,), jnp.int32),
        pltpu.VMEM((IDXN, DH), jnp.float32),
        pltpu.VMEM((ZCH, DH), jnp.float32),
        pltpu.VMEM_SHARED((N_PAD, DH), jnp.float32),
        pltpu.SemaphoreType.DMA,
    ],
    compiler_params=_sc_params,
)
def _sc_agg(hs0_hbm, hs1_hbm, src_hbm, dst_hbm, acc_hbm,
            src_v, dst_v, rows_v, stage_v, acc_sp, sem):
  cid = lax.axis_index("c")
  sid = lax.axis_index("s")

  _fill2d(stage_v, ZCH // 2, 0.0)
  for k in range(8):
    pltpu.sync_copy(stage_v, acc_sp.at[pl.ds(sid * SEG + k * ZCH, ZCH)])
  plsc.subcore_barrier()

  tbase = sid * EPT

  def chunk_core(ci, hs_hbm):
    r0 = tbase + ci * IDXN
    pltpu.sync_copy(src_hbm.at[pl.ds(r0, IDXN)], src_v)
    pltpu.sync_copy(dst_hbm.at[pl.ds(r0, IDXN)], dst_v)
    pltpu.async_copy(hs_hbm.at[src_v], rows_v, sem).wait()
    pltpu.sync_copy(rows_v, acc_sp.at[dst_v], add=True)

  @pl.when(cid == 0)
  def _():
    def body(ci, _):
      chunk_core(ci, hs0_hbm)
      return 0
    lax.fori_loop(0, NCHUNK, body, 0)

  @pl.when(cid == 1)
  def _():
    def body(ci, _):
      chunk_core(ci, hs1_hbm)
      return 0
    lax.fori_loop(0, NCHUNK, body, 0)

  plsc.subcore_barrier()

  pltpu.sync_copy(acc_sp.at[pl.ds(sid * SEG, SEG)],
                  acc_hbm.at[cid, pl.ds(sid * SEG, SEG)])


def _tc_z_body(x_ref, w_ref, z_ref):
  z_ref[...] = jnp.dot(x_ref[...], w_ref[...],
                       preferred_element_type=jnp.float32)


def _tc_scale_body(z_ref, d0_ref, d1_ref, hs0_ref, hs1_ref, dis_ref):
  d = d0_ref[...] + d1_ref[...] + 1.0
  dis = jnp.where(d > 0, lax.rsqrt(jnp.maximum(d, 1e-12)), 0.0)
  hs = z_ref[...] * dis
  hs0_ref[...] = hs[:, :DH]
  hs1_ref[...] = hs[:, DH:]
  dis_ref[...] = dis


def _tc_mid_body(a0_ref, a1_ref, hs0_ref, hs1_ref, dis_ref, w_ref, b_ref,
                 o0_ref, o1_ref):
  dis = dis_ref[...]
  a = jnp.concatenate([a0_ref[...], a1_ref[...]], axis=1)
  hs = jnp.concatenate([hs0_ref[...], hs1_ref[...]], axis=1)
  h = dis * (a + hs) + b_ref[...]
  h = jnp.maximum(h, 0.0)
  z = jnp.dot(h, w_ref[...], preferred_element_type=jnp.float32) * dis
  o0_ref[...] = z[:, :DH]
  o1_ref[...] = z[:, DH:]


def _tc_last_body(a0_ref, a1_ref, hs0_ref, hs1_ref, dis_ref, b_ref, out_ref):
  a = jnp.concatenate([a0_ref[...], a1_ref[...]], axis=1)
  hs = jnp.concatenate([hs0_ref[...], hs1_ref[...]], axis=1)
  out_ref[...] = dis_ref[...] * (a + hs) + b_ref[...]


def _rows_spec(width):
  return pl.BlockSpec((ROW_BLK, width), lambda i: (i, 0))


def _full_spec(shape):
  return pl.BlockSpec(shape, lambda i: tuple(0 for _ in shape))


_tc_params = pltpu.CompilerParams(dimension_semantics=("arbitrary",))

_tc_z = pl.pallas_call(
    _tc_z_body,
    grid=(TC_GRID,),
    in_specs=[_rows_spec(DIN), _full_spec((DIN, DOUT))],
    out_specs=_rows_spec(DOUT),
    out_shape=jax.ShapeDtypeStruct((N_PAD, DOUT), jnp.float32),
    compiler_params=_tc_params,
)

_tc_scale = pl.pallas_call(
    _tc_scale_body,
    grid=(TC_GRID,),
    in_specs=[_rows_spec(DOUT), _rows_spec(1), _rows_spec(1)],
    out_specs=[_rows_spec(DH), _rows_spec(DH), _rows_spec(1)],
    out_shape=[
        jax.ShapeDtypeStruct((N_PAD, DH), jnp.float32),
        jax.ShapeDtypeStruct((N_PAD, DH), jnp.float32),
        jax.ShapeDtypeStruct((N_PAD, 1), jnp.float32),
    ],
    compiler_params=_tc_params,
)

_tc_mid = pl.pallas_call(
    _tc_mid_body,
    grid=(TC_GRID,),
    in_specs=[
        _rows_spec(DH),
        _rows_spec(DH),
        _rows_spec(DH),
        _rows_spec(DH),
        _rows_spec(1),
        _full_spec((DOUT, DOUT)),
        _full_spec((1, DOUT)),
    ],
    out_specs=[_rows_spec(DH), _rows_spec(DH)],
    out_shape=[
        jax.ShapeDtypeStruct((N_PAD, DH), jnp.float32),
        jax.ShapeDtypeStruct((N_PAD, DH), jnp.float32),
    ],
    compiler_params=_tc_params,
)

_tc_last = pl.pallas_call(
    _tc_last_body,
    grid=(TC_GRID,),
    in_specs=[
        _rows_spec(DH),
        _rows_spec(DH),
        _rows_spec(DH),
        _rows_spec(DH),
        _rows_spec(1),
        _full_spec((1, DOUT)),
    ],
    out_specs=_rows_spec(DOUT),
    out_shape=jax.ShapeDtypeStruct((N_PAD, DOUT), jnp.float32),
    compiler_params=_tc_params,
)


@jax.jit
def kernel(x, edge_index, W1, b1, W2, b2):
  pad_e = E_PAD - E
  src = jnp.concatenate([edge_index[0], jnp.full((pad_e,), N, jnp.int32)])
  dst = jnp.concatenate([edge_index[1], jnp.full((pad_e,), N, jnp.int32)])
  x_pad = jnp.pad(x, ((0, N_PAD - N), (0, 0)))

  deg = _sc_degree(dst)
  z1 = _tc_z(x_pad, W1)
  d0 = deg[0].reshape(N_PAD, 1)
  d1 = deg[1].reshape(N_PAD, 1)

  hs1a, hs1b, dis = _tc_scale(z1, d0, d1)

  acc1 = _sc_agg(hs1a, hs1b, src, dst)
  hs2a, hs2b = _tc_mid(acc1[0], acc1[1], hs1a, hs1b, dis, W2,
                       b1.reshape(1, DOUT))

  acc2 = _sc_agg(hs2a, hs2b, src, dst)
  out = _tc_last(acc2[0], acc2[1], hs2a, hs2b, dis, b2.reshape(1, DOUT))
  return out[:N]


# speedup vs baseline: 1.1867x; 1.1257x over previous
import functools

import jax
import jax.numpy as jnp
from jax import lax
from jax.experimental import pallas as pl
from jax.experimental.pallas import tpu as pltpu
from jax.experimental.pallas import tpu_sc as plsc

N = 100000
E = 3200000
DIN = 128
DOUT = 16
DH = DOUT // 2

NC = 2
NS = 16

ROW_BLK = 2048
N_PAD = 100352
TC_GRID = N_PAD // ROW_BLK
SEG = N_PAD // NS
ZCH = SEG // 8

E_PAD = 2048 * 32 * 50
IDXN = 2048
EPT = E_PAD // NS
NCHUNK = EPT // IDXN
DIDXN = 4096
DEPT = E_PAD // (NC * NS)
DNCHUNK = DEPT // DIDXN

_mesh = plsc.VectorSubcoreMesh(
    core_axis_name="c", subcore_axis_name="s", num_cores=NC, num_subcores=NS
)
_sc_params = pltpu.CompilerParams(use_tc_tiling_on_sc=False)


def _fill1d(ref, nvec, value):
  v = jnp.full((16,), value, jnp.float32)

  def body(i, _):
    ref[pl.ds(i * 16, 16)] = v
    return 0

  lax.fori_loop(0, nvec, body, 0)


def _fill2d(ref, nvec, value):
  v = jnp.full((2, 8), value, jnp.float32)

  def body(i, _):
    ref[pl.ds(i * 2, 2), :] = v
    return 0

  lax.fori_loop(0, nvec, body, 0)


@functools.partial(
    pl.kernel,
    out_type=jax.ShapeDtypeStruct((NC, N_PAD), jnp.float32),
    mesh=_mesh,
    scratch_types=[
        pltpu.VMEM((DIDXN,), jnp.int32),
        pltpu.VMEM((DIDXN,), jnp.float32),
        pltpu.VMEM((SEG,), jnp.float32),
        pltpu.VMEM_SHARED((N_PAD,), jnp.float32),
    ],
    compiler_params=_sc_params,
)
def _sc_degree(dst_hbm, deg_hbm, idx_v, ones_v, stage_v, deg_sp):
  cid = lax.axis_index("c")
  sid = lax.axis_index("s")
  wid = cid * NS + sid

  _fill1d(ones_v, DIDXN // 16, 1.0)
  _fill1d(stage_v, SEG // 16, 0.0)
  pltpu.sync_copy(stage_v, deg_sp.at[pl.ds(sid * SEG, SEG)])
  plsc.subcore_barrier()

  tbase = wid * DEPT

  def chunk_body(ci, _):
    pltpu.sync_copy(dst_hbm.at[pl.ds(tbase + ci * DIDXN, DIDXN)], idx_v)
    pltpu.sync_copy(ones_v, deg_sp.at[idx_v], add=True)
    return 0

  lax.fori_loop(0, DNCHUNK, chunk_body, 0)
  plsc.subcore_barrier()

  pltpu.sync_copy(deg_sp.at[pl.ds(sid * SEG, SEG)],
                  deg_hbm.at[cid, pl.ds(sid * SEG, SEG)])


@functools.partial(
    pl.kernel,
    out_type=jax.ShapeDtypeStruct((NC, N_PAD, DH), jnp.float32),
    mesh=_mesh,
    scratch_types=[
        pltpu.VMEM((2, IDXN), jnp.int32),
        pltpu.VMEM((2, IDXN), jnp.int32),
        pltpu.VMEM((2, IDXN, DH), jnp.float32),
        pltpu.VMEM((ZCH, DH), jnp.float32),
        pltpu.VMEM_SHARED((N_PAD, DH), jnp.float32),
        pltpu.SemaphoreType.DMA,
        pltpu.SemaphoreType.DMA,
        pltpu.SemaphoreType.DMA,
        pltpu.SemaphoreType.DMA,
    ],
    compiler_params=_sc_params,
)
def _sc_agg(hs0_hbm, hs1_hbm, src_hbm, dst_hbm, acc_hbm,
            src_v, dst_v, rows_v, stage_v, acc_sp, g0, g1, s0, s1):
  cid = lax.axis_index("c")
  sid = lax.axis_index("s")
  gsem = (g0, g1)
  ssem = (s0, s1)

  _fill2d(stage_v, ZCH // 2, 0.0)
  for k in range(8):
    pltpu.sync_copy(stage_v, acc_sp.at[pl.ds(sid * SEG + k * ZCH, ZCH)])
  plsc.subcore_barrier()

  tbase = sid * EPT

  def run_core(hs_hbm):
    def gather(c, p):
      pltpu.sync_copy(src_hbm.at[pl.ds(tbase + c * IDXN, IDXN)],
                      src_v.at[p])
      pltpu.sync_copy(dst_hbm.at[pl.ds(tbase + c * IDXN, IDXN)],
                      dst_v.at[p])
      pltpu.async_copy(hs_hbm.at[src_v.at[p]], rows_v.at[p], gsem[p])

    def gwait(p):
      pltpu.make_async_copy(
          hs_hbm.at[src_v.at[p]], rows_v.at[p], gsem[p]).wait()

    def scat(p):
      pltpu.async_copy(rows_v.at[p], acc_sp.at[dst_v.at[p]], ssem[p],
                       add=True)

    def swait(p):
      pltpu.make_async_copy(
          rows_v.at[p], acc_sp.at[dst_v.at[p]], ssem[p]).wait()

    gather(0, 0)
    gather(1, 1)

    def body(i, _):
      c = 2 * i
      gwait(0)
      scat(0)
      gwait(1)
      scat(1)

      @pl.when(i < NCHUNK // 2 - 1)
      def _():
        swait(0)
        gather(c + 2, 0)
        swait(1)
        gather(c + 3, 1)
      return 0

    lax.fori_loop(0, NCHUNK // 2, body, 0)
    swait(0)
    swait(1)

  @pl.when(cid == 0)
  def _():
    run_core(hs0_hbm)

  @pl.when(cid == 1)
  def _():
    run_core(hs1_hbm)

  plsc.subcore_barrier()

  pltpu.sync_copy(acc_sp.at[pl.ds(sid * SEG, SEG)],
                  acc_hbm.at[cid, pl.ds(sid * SEG, SEG)])


def _tc_z_body(x_ref, w_ref, z_ref):
  z_ref[...] = jnp.dot(x_ref[...], w_ref[...],
                       preferred_element_type=jnp.float32)


def _tc_scale_body(z_ref, d0_ref, d1_ref, hs0_ref, hs1_ref, dis_ref):
  d = d0_ref[...] + d1_ref[...] + 1.0
  dis = jnp.where(d > 0, lax.rsqrt(jnp.maximum(d, 1e-12)), 0.0)
  hs = z_ref[...] * dis
  hs0_ref[...] = hs[:, :DH]
  hs1_ref[...] = hs[:, DH:]
  dis_ref[...] = dis


def _tc_mid_body(a0_ref, a1_ref, hs0_ref, hs1_ref, dis_ref, w_ref, b_ref,
                 o0_ref, o1_ref):
  dis = dis_ref[...]
  a = jnp.concatenate([a0_ref[...], a1_ref[...]], axis=1)
  hs = jnp.concatenate([hs0_ref[...], hs1_ref[...]], axis=1)
  h = dis * (a + hs) + b_ref[...]
  h = jnp.maximum(h, 0.0)
  z = jnp.dot(h, w_ref[...], preferred_element_type=jnp.float32) * dis
  o0_ref[...] = z[:, :DH]
  o1_ref[...] = z[:, DH:]


def _tc_last_body(a0_ref, a1_ref, hs0_ref, hs1_ref, dis_ref, b_ref, out_ref):
  a = jnp.concatenate([a0_ref[...], a1_ref[...]], axis=1)
  hs = jnp.concatenate([hs0_ref[...], hs1_ref[...]], axis=1)
  out_ref[...] = dis_ref[...] * (a + hs) + b_ref[...]


def _rows_spec(width):
  return pl.BlockSpec((ROW_BLK, width), lambda i: (i, 0))


def _full_spec(shape):
  return pl.BlockSpec(shape, lambda i: tuple(0 for _ in shape))


_tc_params = pltpu.CompilerParams(dimension_semantics=("arbitrary",))

_tc_z = pl.pallas_call(
    _tc_z_body,
    grid=(TC_GRID,),
    in_specs=[_rows_spec(DIN), _full_spec((DIN, DOUT))],
    out_specs=_rows_spec(DOUT),
    out_shape=jax.ShapeDtypeStruct((N_PAD, DOUT), jnp.float32),
    compiler_params=_tc_params,
)

_tc_scale = pl.pallas_call(
    _tc_scale_body,
    grid=(TC_GRID,),
    in_specs=[_rows_spec(DOUT), _rows_spec(1), _rows_spec(1)],
    out_specs=[_rows_spec(DH), _rows_spec(DH), _rows_spec(1)],
    out_shape=[
        jax.ShapeDtypeStruct((N_PAD, DH), jnp.float32),
        jax.ShapeDtypeStruct((N_PAD, DH), jnp.float32),
        jax.ShapeDtypeStruct((N_PAD, 1), jnp.float32),
    ],
    compiler_params=_tc_params,
)

_tc_mid = pl.pallas_call(
    _tc_mid_body,
    grid=(TC_GRID,),
    in_specs=[
        _rows_spec(DH),
        _rows_spec(DH),
        _rows_spec(DH),
        _rows_spec(DH),
        _rows_spec(1),
        _full_spec((DOUT, DOUT)),
        _full_spec((1, DOUT)),
    ],
    out_specs=[_rows_spec(DH), _rows_spec(DH)],
    out_shape=[
        jax.ShapeDtypeStruct((N_PAD, DH), jnp.float32),
        jax.ShapeDtypeStruct((N_PAD, DH), jnp.float32),
    ],
    compiler_params=_tc_params,
)

_tc_last = pl.pallas_call(
    _tc_last_body,
    grid=(TC_GRID,),
    in_specs=[
        _rows_spec(DH),
        _rows_spec(DH),
        _rows_spec(DH),
        _rows_spec(DH),
        _rows_spec(1),
        _full_spec((1, DOUT)),
    ],
    out_specs=_rows_spec(DOUT),
    out_shape=jax.ShapeDtypeStruct((N_PAD, DOUT), jnp.float32),
    compiler_params=_tc_params,
)


@jax.jit
def kernel(x, edge_index, W1, b1, W2, b2):
  pad_e = E_PAD - E
  src = jnp.concatenate([edge_index[0], jnp.full((pad_e,), N, jnp.int32)])
  dst = jnp.concatenate([edge_index[1], jnp.full((pad_e,), N, jnp.int32)])
  x_pad = jnp.pad(x, ((0, N_PAD - N), (0, 0)))

  deg = _sc_degree(dst)
  z1 = _tc_z(x_pad, W1)
  d0 = deg[0].reshape(N_PAD, 1)
  d1 = deg[1].reshape(N_PAD, 1)

  hs1a, hs1b, dis = _tc_scale(z1, d0, d1)

  acc1 = _sc_agg(hs1a, hs1b, src, dst)
  hs2a, hs2b = _tc_mid(acc1[0], acc1[1], hs1a, hs1b, dis, W2,
                       b1.reshape(1, DOUT))

  acc2 = _sc_agg(hs2a, hs2b, src, dst)
  out = _tc_last(acc2[0], acc2[1], hs2a, hs2b, dis, b2.reshape(1, DOUT))
  return out[:N]


# speedup vs baseline: 1.2665x; 1.0673x over previous
import functools

import jax
import jax.numpy as jnp
from jax import lax
from jax.experimental import pallas as pl
from jax.experimental.pallas import tpu as pltpu
from jax.experimental.pallas import tpu_sc as plsc

N = 100000
E = 3200000
DIN = 128
DOUT = 16
DH = DOUT // 2

NC = 2
NS = 16

ROW_BLK = 2048
N_PAD = 100352
TC_GRID = N_PAD // ROW_BLK
SEG = N_PAD // NS
ZCH = SEG // 8

E_PAD = 2048 * 32 * 50
IDXN = 2048
EPT = E_PAD // NS
NCHUNK = EPT // IDXN
DIDXN = 4096
DEPT = E_PAD // (NC * NS)
DNCHUNK = DEPT // DIDXN

_mesh = plsc.VectorSubcoreMesh(
    core_axis_name="c", subcore_axis_name="s", num_cores=NC, num_subcores=NS
)
_sc_params = pltpu.CompilerParams(use_tc_tiling_on_sc=False)


def _fill1d(ref, nvec, value):
  v = jnp.full((16,), value, jnp.float32)

  def body(i, _):
    ref[pl.ds(i * 16, 16)] = v
    return 0

  lax.fori_loop(0, nvec, body, 0)


def _fill2d(ref, nvec, value):
  v = jnp.full((2, 8), value, jnp.float32)

  def body(i, _):
    ref[pl.ds(i * 2, 2), :] = v
    return 0

  lax.fori_loop(0, nvec, body, 0)


@functools.partial(
    pl.kernel,
    out_type=jax.ShapeDtypeStruct((NC, N_PAD), jnp.float32),
    mesh=_mesh,
    scratch_types=[
        pltpu.VMEM((DIDXN,), jnp.int32),
        pltpu.VMEM((DIDXN,), jnp.float32),
        pltpu.VMEM((SEG,), jnp.float32),
        pltpu.VMEM_SHARED((N_PAD,), jnp.float32),
    ],
    compiler_params=_sc_params,
)
def _sc_degree(dst_hbm, deg_hbm, idx_v, ones_v, stage_v, deg_sp):
  cid = lax.axis_index("c")
  sid = lax.axis_index("s")
  wid = cid * NS + sid

  _fill1d(ones_v, DIDXN // 16, 1.0)
  _fill1d(stage_v, SEG // 16, 0.0)
  pltpu.sync_copy(stage_v, deg_sp.at[pl.ds(sid * SEG, SEG)])
  plsc.subcore_barrier()

  tbase = wid * DEPT

  def chunk_body(ci, _):
    pltpu.sync_copy(dst_hbm.at[pl.ds(tbase + ci * DIDXN, DIDXN)], idx_v)
    pltpu.sync_copy(ones_v, deg_sp.at[idx_v], add=True)
    return 0

  lax.fori_loop(0, DNCHUNK, chunk_body, 0)
  plsc.subcore_barrier()

  pltpu.sync_copy(deg_sp.at[pl.ds(sid * SEG, SEG)],
                  deg_hbm.at[cid, pl.ds(sid * SEG, SEG)])


@functools.partial(
    pl.kernel,
    out_type=jax.ShapeDtypeStruct((NC, N_PAD, DH), jnp.float32),
    mesh=_mesh,
    scratch_types=[
        pltpu.VMEM((2, IDXN), jnp.int32),
        pltpu.VMEM((2, IDXN), jnp.int32),
        pltpu.VMEM((2, IDXN, DH), jnp.float32),
        pltpu.VMEM((ZCH, DH), jnp.float32),
        pltpu.VMEM_SHARED((N_PAD, DH), jnp.float32),
        pltpu.SemaphoreType.DMA,
        pltpu.SemaphoreType.DMA,
        pltpu.SemaphoreType.DMA,
        pltpu.SemaphoreType.DMA,
    ],
    compiler_params=_sc_params,
)
def _sc_agg(hs_hbm, src_hbm, dst_hbm, acc_hbm,
            src_v, dst_v, rows_v, stage_v, acc_sp, g0, g1, s0, s1):
  cid = lax.axis_index("c")
  sid = lax.axis_index("s")
  gsem = (g0, g1)
  ssem = (s0, s1)

  _fill2d(stage_v, ZCH // 2, 0.0)
  for k in range(8):
    pltpu.sync_copy(stage_v, acc_sp.at[pl.ds(sid * SEG + k * ZCH, ZCH)])
  plsc.subcore_barrier()

  tbase = sid * EPT
  hs_core = hs_hbm.at[cid]

  def run_core(hs_hbm):
    def gather(c, p):
      pltpu.sync_copy(src_hbm.at[pl.ds(tbase + c * IDXN, IDXN)],
                      src_v.at[p])
      pltpu.sync_copy(dst_hbm.at[pl.ds(tbase + c * IDXN, IDXN)],
                      dst_v.at[p])
      pltpu.async_copy(hs_hbm.at[src_v.at[p]], rows_v.at[p], gsem[p])

    def gwait(p):
      pltpu.make_async_copy(
          hs_hbm.at[src_v.at[p]], rows_v.at[p], gsem[p]).wait()

    def scat(p):
      pltpu.async_copy(rows_v.at[p], acc_sp.at[dst_v.at[p]], ssem[p],
                       add=True)

    def swait(p):
      pltpu.make_async_copy(
          rows_v.at[p], acc_sp.at[dst_v.at[p]], ssem[p]).wait()

    gather(0, 0)
    gather(1, 1)

    def body(i, _):
      c = 2 * i
      gwait(0)
      scat(0)
      gwait(1)
      scat(1)

      @pl.when(i < NCHUNK // 2 - 1)
      def _():
        swait(0)
        gather(c + 2, 0)
        swait(1)
        gather(c + 3, 1)
      return 0

    lax.fori_loop(0, NCHUNK // 2, body, 0)
    swait(0)
    swait(1)

  run_core(hs_core)

  plsc.subcore_barrier()

  pltpu.sync_copy(acc_sp.at[pl.ds(sid * SEG, SEG)],
                  acc_hbm.at[cid, pl.ds(sid * SEG, SEG)])


def _tc_z_body(x_ref, w_ref, z_ref):
  z_ref[...] = jnp.dot(x_ref[...], w_ref[...],
                       preferred_element_type=jnp.float32)


def _tc_scale_body(z_ref, d0_ref, d1_ref, hs_ref, dis_ref):
  d = d0_ref[...] + d1_ref[...] + 1.0
  dis = jnp.where(d > 0, lax.rsqrt(jnp.maximum(d, 1e-12)), 0.0)
  hs = z_ref[...] * dis
  hs_ref[0, :, :] = hs[:, :DH]
  hs_ref[1, :, :] = hs[:, DH:]
  dis_ref[...] = dis


def _tc_mid_body(a_ref, hs_ref, dis_ref, w_ref, b_ref, o_ref):
  dis = dis_ref[...]
  a = jnp.concatenate([a_ref[0, :, :], a_ref[1, :, :]], axis=1)
  hs = jnp.concatenate([hs_ref[0, :, :], hs_ref[1, :, :]], axis=1)
  h = dis * (a + hs) + b_ref[...]
  h = jnp.maximum(h, 0.0)
  z = jnp.dot(h, w_ref[...], preferred_element_type=jnp.float32) * dis
  o_ref[0, :, :] = z[:, :DH]
  o_ref[1, :, :] = z[:, DH:]


def _tc_last_body(a_ref, hs_ref, dis_ref, b_ref, out_ref):
  a = jnp.concatenate([a_ref[0, :, :], a_ref[1, :, :]], axis=1)
  hs = jnp.concatenate([hs_ref[0, :, :], hs_ref[1, :, :]], axis=1)
  out_ref[...] = dis_ref[...] * (a + hs) + b_ref[...]


def _rows_spec(width):
  return pl.BlockSpec((ROW_BLK, width), lambda i: (i, 0))


def _full_spec(shape):
  return pl.BlockSpec(shape, lambda i: tuple(0 for _ in shape))


_tc_params = pltpu.CompilerParams(dimension_semantics=("arbitrary",))

_tc_z = pl.pallas_call(
    _tc_z_body,
    grid=(TC_GRID,),
    in_specs=[_rows_spec(DIN), _full_spec((DIN, DOUT))],
    out_specs=_rows_spec(DOUT),
    out_shape=jax.ShapeDtypeStruct((N_PAD, DOUT), jnp.float32),
    compiler_params=_tc_params,
)

def _split_spec():
  return pl.BlockSpec((2, ROW_BLK, DH), lambda i: (0, i, 0))


_tc_scale = pl.pallas_call(
    _tc_scale_body,
    grid=(TC_GRID,),
    in_specs=[_rows_spec(DOUT), _rows_spec(1), _rows_spec(1)],
    out_specs=[_split_spec(), _rows_spec(1)],
    out_shape=[
        jax.ShapeDtypeStruct((2, N_PAD, DH), jnp.float32),
        jax.ShapeDtypeStruct((N_PAD, 1), jnp.float32),
    ],
    compiler_params=_tc_params,
)

_tc_mid = pl.pallas_call(
    _tc_mid_body,
    grid=(TC_GRID,),
    in_specs=[
        _split_spec(),
        _split_spec(),
        _rows_spec(1),
        _full_spec((DOUT, DOUT)),
        _full_spec((1, DOUT)),
    ],
    out_specs=_split_spec(),
    out_shape=jax.ShapeDtypeStruct((2, N_PAD, DH), jnp.float32),
    compiler_params=_tc_params,
)

_tc_last = pl.pallas_call(
    _tc_last_body,
    grid=(TC_GRID,),
    in_specs=[
        _split_spec(),
        _split_spec(),
        _rows_spec(1),
        _full_spec((1, DOUT)),
    ],
    out_specs=_rows_spec(DOUT),
    out_shape=jax.ShapeDtypeStruct((N_PAD, DOUT), jnp.float32),
    compiler_params=_tc_params,
)


@jax.jit
def kernel(x, edge_index, W1, b1, W2, b2):
  pad_e = E_PAD - E
  src = jnp.concatenate([edge_index[0], jnp.full((pad_e,), N, jnp.int32)])
  dst = jnp.concatenate([edge_index[1], jnp.full((pad_e,), N, jnp.int32)])
  x_pad = jnp.pad(x, ((0, N_PAD - N), (0, 0)))

  deg = _sc_degree(dst)
  z1 = _tc_z(x_pad, W1)
  d0 = deg[0].reshape(N_PAD, 1)
  d1 = deg[1].reshape(N_PAD, 1)

  hs1, dis = _tc_scale(z1, d0, d1)

  acc1 = _sc_agg(hs1, src, dst)
  hs2 = _tc_mid(acc1, hs1, dis, W2, b1.reshape(1, DOUT))

  acc2 = _sc_agg(hs2, src, dst)
  out = _tc_last(acc2, hs2, dis, b2.reshape(1, DOUT))
  return out[:N]


# speedup vs baseline: 1.2767x; 1.0081x over previous
import functools

import jax
import jax.numpy as jnp
from jax import lax
from jax.experimental import pallas as pl
from jax.experimental.pallas import tpu as pltpu
from jax.experimental.pallas import tpu_sc as plsc

N = 100000
E = 3200000
DIN = 128
DOUT = 16
DH = DOUT // 2

NC = 2
NS = 16

ROW_BLK = 2048
N_PAD = 100352
TC_GRID = N_PAD // ROW_BLK
SEG = N_PAD // NS
ZCH = SEG // 8

E_PAD = 2048 * 32 * 50
IDXN = 3200
EPT = E_PAD // NS
NCHUNK = EPT // IDXN
DIDXN = 4096
DEPT = E_PAD // (NC * NS)
DNCHUNK = DEPT // DIDXN

_mesh = plsc.VectorSubcoreMesh(
    core_axis_name="c", subcore_axis_name="s", num_cores=NC, num_subcores=NS
)
_sc_params = pltpu.CompilerParams(use_tc_tiling_on_sc=False)


def _fill1d(ref, nvec, value):
  v = jnp.full((16,), value, jnp.float32)

  def body(i, _):
    ref[pl.ds(i * 16, 16)] = v
    return 0

  lax.fori_loop(0, nvec, body, 0)


def _fill2d(ref, nvec, value):
  v = jnp.full((2, 8), value, jnp.float32)

  def body(i, _):
    ref[pl.ds(i * 2, 2), :] = v
    return 0

  lax.fori_loop(0, nvec, body, 0)


@functools.partial(
    pl.kernel,
    out_type=jax.ShapeDtypeStruct((NC, N_PAD), jnp.float32),
    mesh=_mesh,
    scratch_types=[
        pltpu.VMEM((DIDXN,), jnp.int32),
        pltpu.VMEM((DIDXN,), jnp.float32),
        pltpu.VMEM((SEG,), jnp.float32),
        pltpu.VMEM_SHARED((N_PAD,), jnp.float32),
    ],
    compiler_params=_sc_params,
)
def _sc_degree(dst_hbm, deg_hbm, idx_v, ones_v, stage_v, deg_sp):
  cid = lax.axis_index("c")
  sid = lax.axis_index("s")
  wid = cid * NS + sid

  _fill1d(ones_v, DIDXN // 16, 1.0)
  _fill1d(stage_v, SEG // 16, 0.0)
  pltpu.sync_copy(stage_v, deg_sp.at[pl.ds(sid * SEG, SEG)])
  plsc.subcore_barrier()

  tbase = wid * DEPT

  def chunk_body(ci, _):
    pltpu.sync_copy(dst_hbm.at[pl.ds(tbase + ci * DIDXN, DIDXN)], idx_v)
    pltpu.sync_copy(ones_v, deg_sp.at[idx_v], add=True)
    return 0

  lax.fori_loop(0, DNCHUNK, chunk_body, 0)
  plsc.subcore_barrier()

  pltpu.sync_copy(deg_sp.at[pl.ds(sid * SEG, SEG)],
                  deg_hbm.at[cid, pl.ds(sid * SEG, SEG)])


@functools.partial(
    pl.kernel,
    out_type=jax.ShapeDtypeStruct((NC, N_PAD, DH), jnp.float32),
    mesh=_mesh,
    scratch_types=[
        pltpu.VMEM((2, IDXN), jnp.int32),
        pltpu.VMEM((2, IDXN), jnp.int32),
        pltpu.VMEM((2, IDXN, DH), jnp.float32),
        pltpu.VMEM((ZCH, DH), jnp.float32),
        pltpu.VMEM_SHARED((N_PAD, DH), jnp.float32),
        pltpu.SemaphoreType.DMA,
        pltpu.SemaphoreType.DMA,
        pltpu.SemaphoreType.DMA,
        pltpu.SemaphoreType.DMA,
    ],
    compiler_params=_sc_params,
)
def _sc_agg(hs_hbm, src_hbm, dst_hbm, acc_hbm,
            src_v, dst_v, rows_v, stage_v, acc_sp, g0, g1, s0, s1):
  cid = lax.axis_index("c")
  sid = lax.axis_index("s")
  gsem = (g0, g1)
  ssem = (s0, s1)

  _fill2d(stage_v, ZCH // 2, 0.0)
  for k in range(8):
    pltpu.sync_copy(stage_v, acc_sp.at[pl.ds(sid * SEG + k * ZCH, ZCH)])
  plsc.subcore_barrier()

  tbase = sid * EPT
  hs_core = hs_hbm.at[cid]

  def run_core(hs_hbm):
    def gather(c, p):
      pltpu.sync_copy(src_hbm.at[pl.ds(tbase + c * IDXN, IDXN)],
                      src_v.at[p])
      pltpu.sync_copy(dst_hbm.at[pl.ds(tbase + c * IDXN, IDXN)],
                      dst_v.at[p])
      pltpu.async_copy(hs_hbm.at[src_v.at[p]], rows_v.at[p], gsem[p])

    def gwait(p):
      pltpu.make_async_copy(
          hs_hbm.at[src_v.at[p]], rows_v.at[p], gsem[p]).wait()

    def scat(p):
      pltpu.async_copy(rows_v.at[p], acc_sp.at[dst_v.at[p]], ssem[p],
                       add=True)

    def swait(p):
      pltpu.make_async_copy(
          rows_v.at[p], acc_sp.at[dst_v.at[p]], ssem[p]).wait()

    gather(0, 0)
    gather(1, 1)

    def body(i, _):
      c = 2 * i
      gwait(0)
      scat(0)
      gwait(1)
      scat(1)

      @pl.when(i < NCHUNK // 2 - 1)
      def _():
        swait(0)
        gather(c + 2, 0)
        swait(1)
        gather(c + 3, 1)
      return 0

    lax.fori_loop(0, NCHUNK // 2, body, 0)
    swait(0)
    swait(1)

  run_core(hs_core)

  plsc.subcore_barrier()

  pltpu.sync_copy(acc_sp.at[pl.ds(sid * SEG, SEG)],
                  acc_hbm.at[cid, pl.ds(sid * SEG, SEG)])


def _tc_z_body(x_ref, w_ref, z_ref):
  z_ref[...] = jnp.dot(x_ref[...], w_ref[...],
                       preferred_element_type=jnp.float32)


def _tc_scale_body(z_ref, d0_ref, d1_ref, hs_ref, dis_ref):
  d = d0_ref[...] + d1_ref[...] + 1.0
  dis = jnp.where(d > 0, lax.rsqrt(jnp.maximum(d, 1e-12)), 0.0)
  hs = z_ref[...] * dis
  hs_ref[0, :, :] = hs[:, :DH]
  hs_ref[1, :, :] = hs[:, DH:]
  dis_ref[...] = dis


def _tc_mid_body(a_ref, hs_ref, dis_ref, w_ref, b_ref, o_ref):
  dis = dis_ref[...]
  a = jnp.concatenate([a_ref[0, :, :], a_ref[1, :, :]], axis=1)
  hs = jnp.concatenate([hs_ref[0, :, :], hs_ref[1, :, :]], axis=1)
  h = dis * (a + hs) + b_ref[...]
  h = jnp.maximum(h, 0.0)
  z = jnp.dot(h, w_ref[...], preferred_element_type=jnp.float32) * dis
  o_ref[0, :, :] = z[:, :DH]
  o_ref[1, :, :] = z[:, DH:]


def _tc_last_body(a_ref, hs_ref, dis_ref, b_ref, out_ref):
  a = jnp.concatenate([a_ref[0, :, :], a_ref[1, :, :]], axis=1)
  hs = jnp.concatenate([hs_ref[0, :, :], hs_ref[1, :, :]], axis=1)
  out_ref[...] = dis_ref[...] * (a + hs) + b_ref[...]


def _rows_spec(width):
  return pl.BlockSpec((ROW_BLK, width), lambda i: (i, 0))


def _full_spec(shape):
  return pl.BlockSpec(shape, lambda i: tuple(0 for _ in shape))


_tc_params = pltpu.CompilerParams(dimension_semantics=("arbitrary",))

_tc_z = pl.pallas_call(
    _tc_z_body,
    grid=(TC_GRID,),
    in_specs=[_rows_spec(DIN), _full_spec((DIN, DOUT))],
    out_specs=_rows_spec(DOUT),
    out_shape=jax.ShapeDtypeStruct((N_PAD, DOUT), jnp.float32),
    compiler_params=_tc_params,
)

def _split_spec():
  return pl.BlockSpec((2, ROW_BLK, DH), lambda i: (0, i, 0))


_tc_scale = pl.pallas_call(
    _tc_scale_body,
    grid=(TC_GRID,),
    in_specs=[_rows_spec(DOUT), _rows_spec(1), _rows_spec(1)],
    out_specs=[_split_spec(), _rows_spec(1)],
    out_shape=[
        jax.ShapeDtypeStruct((2, N_PAD, DH), jnp.float32),
        jax.ShapeDtypeStruct((N_PAD, 1), jnp.float32),
    ],
    compiler_params=_tc_params,
)

_tc_mid = pl.pallas_call(
    _tc_mid_body,
    grid=(TC_GRID,),
    in_specs=[
        _split_spec(),
        _split_spec(),
        _rows_spec(1),
        _full_spec((DOUT, DOUT)),
        _full_spec((1, DOUT)),
    ],
    out_specs=_split_spec(),
    out_shape=jax.ShapeDtypeStruct((2, N_PAD, DH), jnp.float32),
    compiler_params=_tc_params,
)

_tc_last = pl.pallas_call(
    _tc_last_body,
    grid=(TC_GRID,),
    in_specs=[
        _split_spec(),
        _split_spec(),
        _rows_spec(1),
        _full_spec((1, DOUT)),
    ],
    out_specs=_rows_spec(DOUT),
    out_shape=jax.ShapeDtypeStruct((N_PAD, DOUT), jnp.float32),
    compiler_params=_tc_params,
)


@jax.jit
def kernel(x, edge_index, W1, b1, W2, b2):
  pad_e = E_PAD - E
  src = jnp.concatenate([edge_index[0], jnp.full((pad_e,), N, jnp.int32)])
  dst = jnp.concatenate([edge_index[1], jnp.full((pad_e,), N, jnp.int32)])
  x_pad = jnp.pad(x, ((0, N_PAD - N), (0, 0)))

  deg = _sc_degree(dst)
  z1 = _tc_z(x_pad, W1)
  d0 = deg[0].reshape(N_PAD, 1)
  d1 = deg[1].reshape(N_PAD, 1)

  hs1, dis = _tc_scale(z1, d0, d1)

  acc1 = _sc_agg(hs1, src, dst)
  hs2 = _tc_mid(acc1, hs1, dis, W2, b1.reshape(1, DOUT))

  acc2 = _sc_agg(hs2, src, dst)
  out = _tc_last(acc2, hs2, dis, b2.reshape(1, DOUT))
  return out[:N]
